# baseline (speedup 1.0000x reference)
import functools
import jax
import jax.numpy as jnp
from jax import lax
from jax.experimental import pallas as pl
from jax.experimental.pallas import tpu as pltpu
from jax.experimental.pallas import tpu_sc as plsc

_B = 128
_NBATCH = 128
_NW = 32
_BPW = _NBATCH // _NW


def _sc_body(a_hbm, w_hbm, sel_hbm, buf, selbuf, isem, wsem, ssem):
    c = lax.axis_index("c")
    s = lax.axis_index("s")
    wid = s * 2 + c
    batch0 = wid * _BPW
    iota = lax.broadcasted_iota(jnp.int32, (16,), 0)
    rot1 = jnp.bitwise_and(iota + 1, 15)
    negramp = [-1.0 - (iota + 16 * j).astype(jnp.float32) for j in range(8)]
    gcols = [iota + 16 * j for j in range(8)]
    zerosf = jnp.zeros((16,), jnp.float32)

    def in_copy(ci, ph):
        return pltpu.make_async_copy(
            a_hbm.at[batch0 + ci], buf.at[ph], isem.at[ph])

    def w_copy(ci, ph):
        return pltpu.make_async_copy(
            buf.at[ph], w_hbm.at[batch0 + ci], wsem.at[ph])

    def s_copy(ci, ph):
        return pltpu.make_async_copy(
            selbuf.at[ph], sel_hbm.at[batch0 + ci], ssem.at[ph])

    def merge(ak, ai, bk, bi, descending):
        tb = (bk > ak) | ((bk == ak) & (bi < ai))
        hk = jnp.where(tb, bk, ak)
        hi = jnp.where(tb, bi, ai)
        return plsc.sort_key_val(hk, hi, descending=descending)

    in_copy(0, 0).start()

    def chunk_body(ci, _):
        ph = jnp.bitwise_and(ci, 1)
        in_copy(ci, ph).wait()

        @pl.when(ci >= 1)
        def _():
            w_copy(ci - 1, 1 - ph).wait()
            s_copy(ci - 1, 1 - ph).wait()

        @pl.when(ci <= _BPW - 2)
        def _():
            in_copy(ci + 1, 1 - ph).start()

        def build(lst, descending):
            if len(lst) == 1:
                key, col = lst[0]
                return plsc.sort_key_val(key, col, descending=descending)
            h = (len(lst) + 1) // 2
            lk, li = build(lst[:h], True)
            rk, rix = build(lst[h:], False)
            return merge(lk, li, rk, rix, descending)

        for g in range(8):
            @plsc.parallel_loop(16 * g, 16 * g + 16, 1, unroll=2)
            def row_body(ri, g=g):
                v = [buf[ph, ri, pl.ds(16 * j, 16)] for j in range(8)]
                m = jnp.maximum(jnp.maximum(jnp.maximum(v[0], v[1]),
                                            jnp.maximum(v[2], v[3])),
                                jnp.maximum(jnp.maximum(v[4], v[5]),
                                            jnp.maximum(v[6], v[7])))
                mx = jnp.max(m)
                e = [jnp.exp(vj - mx) for vj in v]
                t = (((e[0] + e[1]) + (e[2] + e[3])) +
                     (((e[4] + e[5]) + (e[6] + e[7]))))
                ssum = jnp.sum(t)
                for j in range(g + 1, 8):
                    buf[ph, ri, pl.ds(16 * j, 16)] = zerosf
                nodes = []
                for j in range(g + 1):
                    p = e[j] / ssum
                    p2 = p * p
                    p4 = p2 * p2
                    p8 = p4 * p4
                    if j < g:
                        buf[ph, ri, pl.ds(16 * j, 16)] = p8
                        key = p8
                    else:
                        masked = gcols[g] > ri
                        buf[ph, ri, pl.ds(16 * g, 16)] = jnp.where(
                            masked, 0.0, p8)
                        key = jnp.where(masked, negramp[g], p8)
                    nodes.append((key, gcols[j]))
                fi = jnp.where(iota == 15, 0, build(nodes, True)[1])
                plsc.store_scatter(
                    selbuf,
                    [jnp.full((16,), ph, jnp.int32), rot1,
                     jnp.full((16,), ri, jnp.int32)],
                    fi)

        w_copy(ci, ph).start()
        s_copy(ci, ph).start()
        return 0

    lax.fori_loop(0, _BPW, chunk_body, 0)
    last = _BPW - 1
    w_copy(last, jnp.bitwise_and(last, 1)).wait()
    s_copy(last, jnp.bitwise_and(last, 1)).wait()


_sc_kernel = pl.kernel(
    _sc_body,
    out_type=(
        jax.ShapeDtypeStruct((_NBATCH, _B, _B), jnp.float32),
        jax.ShapeDtypeStruct((_NBATCH, 16, _B), jnp.int32),
    ),
    mesh=plsc.VectorSubcoreMesh(core_axis_name="c", subcore_axis_name="s",
                                num_cores=2, num_subcores=16),
    scratch_types=[
        pltpu.VMEM((2, _B, _B), jnp.float32),
        pltpu.VMEM((2, 16, _B), jnp.int32),
        pltpu.SemaphoreType.DMA((2,)),
        pltpu.SemaphoreType.DMA((2,)),
        pltpu.SemaphoreType.DMA((2,)),
    ],
    compiler_params=pltpu.CompilerParams(needs_layout_passes=False),
)


def kernel(A_hat, layer_idx, num_query_blocks):
    del layer_idx, num_query_blocks
    walk_state, sel_t = _sc_kernel(A_hat)
    return walk_state, jnp.transpose(sel_t, (0, 2, 1))


# speedup vs baseline: 1.4889x; 1.4889x over previous
import functools
import jax
import jax.numpy as jnp
from jax import lax
from jax.experimental import pallas as pl
from jax.experimental.pallas import tpu as pltpu
from jax.experimental.pallas import tpu_sc as plsc

_B = 128
_NBATCH = 128
_NW = 32
_BPW = _NBATCH // _NW


def _sc_body(a_hbm, w_hbm, sel_hbm, buf, selbuf, isem, wsem, ssem):
    c = lax.axis_index("c")
    s = lax.axis_index("s")
    wid = s * 2 + c
    batch0 = wid * _BPW
    iota = lax.broadcasted_iota(jnp.int32, (16,), 0)
    rot1 = jnp.bitwise_and(iota + 1, 15)
    negramp = [-1.0 - (iota + 16 * j).astype(jnp.float32) for j in range(8)]
    gcols = [iota + 16 * j for j in range(8)]
    zerosf = jnp.zeros((16,), jnp.float32)

    def in_copy(ci, ph):
        return pltpu.make_async_copy(
            a_hbm.at[batch0 + ci], buf.at[ph], isem.at[ph])

    def w_copy(ci, ph):
        return pltpu.make_async_copy(
            buf.at[ph], w_hbm.at[batch0 + ci], wsem.at[ph])

    def s_copy(ci, ph):
        return pltpu.make_async_copy(
            selbuf.at[ph], sel_hbm.at[batch0 + ci], ssem.at[ph])

    def merge(ak, ai, bk, bi, descending):
        tb = (bk > ak) | ((bk == ak) & (bi < ai))
        hk = jnp.where(tb, bk, ak)
        hi = jnp.where(tb, bi, ai)
        return plsc.sort_key_val(hk, hi, descending=descending)

    in_copy(0, 0).start()

    def chunk_body(ci, _):
        ph = jnp.bitwise_and(ci, 1)
        in_copy(ci, ph).wait()

        @pl.when(ci >= 1)
        def _():
            w_copy(ci - 1, 1 - ph).wait()
            s_copy(ci - 1, 1 - ph).wait()

        @pl.when(ci <= _BPW - 2)
        def _():
            in_copy(ci + 1, 1 - ph).start()

        def build(lst, descending):
            if len(lst) == 1:
                key, col = lst[0]
                return plsc.sort_key_val(key, col, descending=descending)
            h = (len(lst) + 1) // 2
            lk, li = build(lst[:h], True)
            rk, rix = build(lst[h:], False)
            return merge(lk, li, rk, rix, descending)

        for nc in (4, 8):
            @plsc.parallel_loop((nc - 4) * 16, nc * 16, 1, unroll=2)
            def row_body(ri, nc=nc):
                v = [buf[ph, ri, pl.ds(16 * j, 16)] for j in range(8)]
                m = jnp.maximum(jnp.maximum(jnp.maximum(v[0], v[1]),
                                            jnp.maximum(v[2], v[3])),
                                jnp.maximum(jnp.maximum(v[4], v[5]),
                                            jnp.maximum(v[6], v[7])))
                mx = jnp.max(m)
                e = [jnp.exp(vj - mx) for vj in v]
                t = (((e[0] + e[1]) + (e[2] + e[3])) +
                     (((e[4] + e[5]) + (e[6] + e[7]))))
                ssum = jnp.sum(t)
                for j in range(nc, 8):
                    buf[ph, ri, pl.ds(16 * j, 16)] = zerosf
                nodes = []
                for j in range(nc):
                    p = e[j] / ssum
                    p2 = p * p
                    p4 = p2 * p2
                    p8 = p4 * p4
                    masked = gcols[j] > ri
                    buf[ph, ri, pl.ds(16 * j, 16)] = jnp.where(masked, 0.0, p8)
                    key = jnp.where(masked, negramp[j], p8)
                    nodes.append((key, gcols[j]))
                fi = jnp.where(iota == 15, 0, build(nodes, True)[1])
                plsc.store_scatter(
                    selbuf,
                    [jnp.full((16,), ph, jnp.int32), rot1,
                     jnp.full((16,), ri, jnp.int32)],
                    fi)

        w_copy(ci, ph).start()
        s_copy(ci, ph).start()
        return 0

    lax.fori_loop(0, _BPW, chunk_body, 0)
    last = _BPW - 1
    w_copy(last, jnp.bitwise_and(last, 1)).wait()
    s_copy(last, jnp.bitwise_and(last, 1)).wait()


_sc_kernel = pl.kernel(
    _sc_body,
    out_type=(
        jax.ShapeDtypeStruct((_NBATCH, _B, _B), jnp.float32),
        jax.ShapeDtypeStruct((_NBATCH, 16, _B), jnp.int32),
    ),
    mesh=plsc.VectorSubcoreMesh(core_axis_name="c", subcore_axis_name="s",
                                num_cores=2, num_subcores=16),
    scratch_types=[
        pltpu.VMEM((2, _B, _B), jnp.float32),
        pltpu.VMEM((2, 16, _B), jnp.int32),
        pltpu.SemaphoreType.DMA((2,)),
        pltpu.SemaphoreType.DMA((2,)),
        pltpu.SemaphoreType.DMA((2,)),
    ],
    compiler_params=pltpu.CompilerParams(needs_layout_passes=False),
)


def kernel(A_hat, layer_idx, num_query_blocks):
    del layer_idx, num_query_blocks
    walk_state, sel_t = _sc_kernel(A_hat)
    return walk_state, jnp.transpose(sel_t, (0, 2, 1))
